# Initial kernel scaffold; baseline (speedup 1.0000x reference)
import jax
import jax.numpy as jnp
from jax.experimental import pallas as pl


def kernel(inputs, W0, W1, W2, W3):
    raise NotImplementedError("write your pallas kernel here")


# speedup vs baseline: 2.1715x; 2.1715x over previous
import functools

import jax
import jax.numpy as jnp
from jax import lax
from jax.experimental import pallas as pl
from jax.experimental.pallas import tpu as pltpu
from jax.experimental.pallas import tpu_sc as plsc

BATCH = 16384
NUM_FIELDS = 4
ATTR_DIM = 64

_info = plsc.get_sparse_core_info()
NC, NS, L = _info.num_cores, _info.num_subcores, _info.num_lanes
NW = NC * NS
BW = BATCH // NW
CHUNK = 128
NCHUNK = BW // CHUNK

_mesh = plsc.VectorSubcoreMesh(core_axis_name="c", subcore_axis_name="s")


@functools.partial(
    pl.kernel,
    mesh=_mesh,
    compiler_params=pltpu.CompilerParams(use_tc_tiling_on_sc=False),
    out_type=jax.ShapeDtypeStruct((BATCH, NUM_FIELDS, ATTR_DIM), jnp.float32),
    scratch_types=[
        pltpu.VMEM((NUM_FIELDS * NCHUNK, CHUNK), jnp.int32),
        pltpu.VMEM((CHUNK, ATTR_DIM), jnp.float32),
        pltpu.SemaphoreType.DMA,
        pltpu.SemaphoreType.DMA,
    ],
)
def _co_embed(i0, i1, i2, i3, w0, w1, w2, w3, out, idx_v, rows, isem, gsem):
    wid = lax.axis_index("s") * NC + lax.axis_index("c")
    base = wid * BW

    cols = [i0, i1, i2, i3]
    for f in range(NUM_FIELDS):
        for c in range(NCHUNK):
            pltpu.async_copy(
                cols[f].at[pl.ds(base + c * CHUNK, CHUNK)],
                idx_v.at[f * NCHUNK + c],
                isem,
            )
    for f in range(NUM_FIELDS):
        for c in range(NCHUNK):
            pltpu.make_async_copy(
                cols[f].at[pl.ds(base + c * CHUNK, CHUNK)],
                idx_v.at[f * NCHUNK + c],
                isem,
            ).wait()

    tables = [w0, w1, w2, w3]
    for f in range(NUM_FIELDS):
        for c in range(NCHUNK):
            pltpu.async_copy(
                tables[f].at[idx_v.at[f * NCHUNK + c]], rows, gsem
            ).wait()
            pltpu.sync_copy(rows, out.at[pl.ds(base + c * CHUNK, CHUNK), f])


def kernel(inputs, W0, W1, W2, W3):
    cols = [inputs[:, f] for f in range(NUM_FIELDS)]
    out = _co_embed(*cols, W0, W1, W2, W3)
    return out.reshape(BATCH, NUM_FIELDS * ATTR_DIM)


# speedup vs baseline: 2.9701x; 1.3678x over previous
import functools

import jax
import jax.numpy as jnp
from jax import lax
from jax.experimental import pallas as pl
from jax.experimental.pallas import tpu as pltpu
from jax.experimental.pallas import tpu_sc as plsc

BATCH = 16384
NUM_FIELDS = 4
ATTR_DIM = 64

_info = plsc.get_sparse_core_info()
NC, NS, L = _info.num_cores, _info.num_subcores, _info.num_lanes
NW = NC * NS
BW = BATCH // NW
CHUNK = 128
NCHUNK = BW // CHUNK
NSTEP = NUM_FIELDS * NCHUNK

_mesh = plsc.VectorSubcoreMesh(core_axis_name="c", subcore_axis_name="s")


@functools.partial(
    pl.kernel,
    mesh=_mesh,
    compiler_params=pltpu.CompilerParams(use_tc_tiling_on_sc=False),
    out_type=jax.ShapeDtypeStruct((BATCH, NUM_FIELDS * ATTR_DIM), jnp.float32),
    scratch_types=[
        pltpu.VMEM((NSTEP, CHUNK), jnp.int32),
        pltpu.VMEM((NSTEP, CHUNK), jnp.int32),
        pltpu.VMEM((2, CHUNK, ATTR_DIM), jnp.float32),
        pltpu.SemaphoreType.DMA,
        pltpu.SemaphoreType.DMA,
        pltpu.SemaphoreType.DMA,
        pltpu.SemaphoreType.DMA,
        pltpu.SemaphoreType.DMA,
    ],
)
def _co_embed(idx, w0, w1, w2, w3, out, ramp_v, idx_v, rows,
              isem, g0, g1, o0, o1):
    wid = lax.axis_index("s") * NC + lax.axis_index("c")
    base = wid * BW
    tables = [w0, w1, w2, w3]
    gsem = [g0, g1]
    osem = [o0, o1]

    lane4 = lax.iota(jnp.int32, L) * NUM_FIELDS
    for k in range(NSTEP):
        f, c = k // NCHUNK, k % NCHUNK
        for g in range(CHUNK // L):
            ramp_v[k, pl.ds(g * L, L)] = lane4 + (
                base * NUM_FIELDS + c * CHUNK * NUM_FIELDS + g * L * NUM_FIELDS + f)

    for k in range(NSTEP):
        pltpu.async_copy(idx.at[ramp_v.at[k]], idx_v.at[k], isem)
    for k in range(NSTEP):
        pltpu.make_async_copy(idx.at[ramp_v.at[k]], idx_v.at[k], isem).wait()

    def out_dst(k):
        f, c = k // NCHUNK, k % NCHUNK
        return out.at[pl.ds(base + c * CHUNK, CHUNK),
                      pl.ds(f * ATTR_DIM, ATTR_DIM)]

    def gather(k):
        f = k // NCHUNK
        pltpu.async_copy(tables[f].at[idx_v.at[k]], rows.at[k % 2],
                         gsem[k % 2])

    def gather_wait(k):
        f = k // NCHUNK
        pltpu.make_async_copy(tables[f].at[idx_v.at[k]], rows.at[k % 2],
                              gsem[k % 2]).wait()

    def writeback(k):
        pltpu.async_copy(rows.at[k % 2], out_dst(k), osem[k % 2])

    def writeback_wait(k):
        pltpu.make_async_copy(rows.at[k % 2], out_dst(k), osem[k % 2]).wait()

    gather(0)
    for k in range(1, NSTEP):
        gather_wait(k - 1)
        writeback(k - 1)
        if k >= 2:
            writeback_wait(k - 2)
        gather(k)
    gather_wait(NSTEP - 1)
    writeback_wait(NSTEP - 2)
    writeback(NSTEP - 1)
    writeback_wait(NSTEP - 1)


def kernel(inputs, W0, W1, W2, W3):
    return _co_embed(inputs.reshape(-1), W0, W1, W2, W3)


# speedup vs baseline: 3.9336x; 1.3244x over previous
import functools

import jax
import jax.numpy as jnp
from jax import lax
from jax.experimental import pallas as pl
from jax.experimental.pallas import tpu as pltpu
from jax.experimental.pallas import tpu_sc as plsc

BATCH = 16384
NUM_FIELDS = 4
ATTR_DIM = 64

_info = plsc.get_sparse_core_info()
NC, NS, L = _info.num_cores, _info.num_subcores, _info.num_lanes
NW = NC * NS
BW = BATCH // NW
CHUNK = 128
NCHUNK = BW // CHUNK
NSTEP = NUM_FIELDS * NCHUNK
NBUF = 4

_mesh = plsc.VectorSubcoreMesh(core_axis_name="c", subcore_axis_name="s")


@functools.partial(
    pl.kernel,
    mesh=_mesh,
    compiler_params=pltpu.CompilerParams(use_tc_tiling_on_sc=False),
    out_type=jax.ShapeDtypeStruct((BATCH, NUM_FIELDS * ATTR_DIM), jnp.float32),
    scratch_types=[
        pltpu.VMEM((NSTEP, CHUNK), jnp.int32),
        pltpu.VMEM((NBUF, CHUNK, ATTR_DIM), jnp.float32),
        pltpu.SemaphoreType.DMA,
        pltpu.SemaphoreType.DMA,
        pltpu.SemaphoreType.DMA,
        pltpu.SemaphoreType.DMA,
        pltpu.SemaphoreType.DMA,
        pltpu.SemaphoreType.DMA,
        pltpu.SemaphoreType.DMA,
        pltpu.SemaphoreType.DMA,
        pltpu.SemaphoreType.DMA,
    ],
)
def _co_embed(idx, w0, w1, w2, w3, out, idx_v, rows,
              isem, g0, g1, g2, g3, o0, o1, o2, o3):
    wid = lax.axis_index("s") * NC + lax.axis_index("c")
    base = wid * BW
    tables = [w0, w1, w2, w3]
    gsem = [g0, g1, g2, g3]
    osem = [o0, o1, o2, o3]

    def idx_src(k):
        f, c = k // NCHUNK, k % NCHUNK
        return idx.at[pl.ds(f * BATCH + base + c * CHUNK, CHUNK)]

    for k in range(NSTEP):
        pltpu.async_copy(idx_src(k), idx_v.at[k], isem)
    for k in range(NSTEP):
        pltpu.make_async_copy(idx_src(k), idx_v.at[k], isem).wait()

    def out_dst(k):
        f, c = k // NCHUNK, k % NCHUNK
        return out.at[pl.ds(base + c * CHUNK, CHUNK),
                      pl.ds(f * ATTR_DIM, ATTR_DIM)]

    def gather(k):
        f = k // NCHUNK
        pltpu.async_copy(tables[f].at[idx_v.at[k]], rows.at[k % NBUF],
                         gsem[k % NBUF])

    def gather_wait(k):
        f = k // NCHUNK
        pltpu.make_async_copy(tables[f].at[idx_v.at[k]], rows.at[k % NBUF],
                              gsem[k % NBUF]).wait()

    def writeback(k):
        pltpu.async_copy(rows.at[k % NBUF], out_dst(k), osem[k % NBUF])

    def writeback_wait(k):
        pltpu.make_async_copy(rows.at[k % NBUF], out_dst(k),
                              osem[k % NBUF]).wait()

    for k in range(NSTEP + NBUF - 1):
        if k < NSTEP:
            if k >= NBUF:
                writeback_wait(k - NBUF)
            gather(k)
        j = k - (NBUF - 1)
        if 0 <= j < NSTEP:
            gather_wait(j)
            writeback(j)
    for j in range(NSTEP - NBUF, NSTEP):
        writeback_wait(j)


def kernel(inputs, W0, W1, W2, W3):
    return _co_embed(inputs.T.reshape(-1), W0, W1, W2, W3)


# speedup vs baseline: 5.2354x; 1.3309x over previous
import functools

import jax
import jax.numpy as jnp
from jax import lax
from jax.experimental import pallas as pl
from jax.experimental.pallas import tpu as pltpu
from jax.experimental.pallas import tpu_sc as plsc

BATCH = 16384
NUM_FIELDS = 4
ATTR_DIM = 64

_info = plsc.get_sparse_core_info()
NC, NS, L = _info.num_cores, _info.num_subcores, _info.num_lanes
NW = NC * NS
BW = BATCH // NW
CHUNK = 128
NCHUNK = BW // CHUNK
NSTEP = NUM_FIELDS * NCHUNK
NBUF = 4

_mesh = plsc.VectorSubcoreMesh(core_axis_name="c", subcore_axis_name="s")


@functools.partial(
    pl.kernel,
    mesh=_mesh,
    compiler_params=pltpu.CompilerParams(use_tc_tiling_on_sc=False),
    out_type=jax.ShapeDtypeStruct((BATCH // 8, 2, 8, 128), jnp.float32),
    scratch_types=[
        pltpu.VMEM((NSTEP, CHUNK), jnp.int32),
        pltpu.VMEM((NBUF, CHUNK, ATTR_DIM), jnp.float32),
        pltpu.SemaphoreType.DMA,
        pltpu.SemaphoreType.DMA,
        pltpu.SemaphoreType.DMA,
        pltpu.SemaphoreType.DMA,
        pltpu.SemaphoreType.DMA,
        pltpu.SemaphoreType.DMA,
        pltpu.SemaphoreType.DMA,
        pltpu.SemaphoreType.DMA,
        pltpu.SemaphoreType.DMA,
    ],
)
def _co_embed(idx, w0, w1, w2, w3, out, idx_v, rows,
              isem, g0, g1, g2, g3, o0, o1, o2, o3):
    wid = lax.axis_index("s") * NC + lax.axis_index("c")
    base = wid * BW
    tables = [w0, w1, w2, w3]
    gsem = [g0, g1, g2, g3]
    osem = [o0, o1, o2, o3]

    def idx_src(k):
        f, c = k // NCHUNK, k % NCHUNK
        return idx.at[pl.ds(f * BATCH + base + c * CHUNK, CHUNK)]

    for k in range(NSTEP):
        pltpu.async_copy(idx_src(k), idx_v.at[k], isem)
    for k in range(NSTEP):
        pltpu.make_async_copy(idx_src(k), idx_v.at[k], isem).wait()

    def out_dst(k, t):
        f, c = k // NCHUNK, k % NCHUNK
        tile0 = (base + c * CHUNK) // 8
        return out.at[tile0 + t, f // 2, :,
                      pl.ds((f % 2) * ATTR_DIM, ATTR_DIM)]

    def gather(k):
        f = k // NCHUNK
        pltpu.async_copy(tables[f].at[idx_v.at[k]], rows.at[k % NBUF],
                         gsem[k % NBUF])

    def gather_wait(k):
        f = k // NCHUNK
        pltpu.make_async_copy(tables[f].at[idx_v.at[k]], rows.at[k % NBUF],
                              gsem[k % NBUF]).wait()

    def writeback(k):
        for t in range(CHUNK // 8):
            pltpu.async_copy(rows.at[k % NBUF, pl.ds(t * 8, 8)],
                             out_dst(k, t), osem[k % NBUF])

    def writeback_wait(k):
        for t in range(CHUNK // 8):
            pltpu.make_async_copy(rows.at[k % NBUF, pl.ds(t * 8, 8)],
                                  out_dst(k, t), osem[k % NBUF]).wait()

    for k in range(NSTEP + NBUF - 1):
        if k < NSTEP:
            if k >= NBUF:
                writeback_wait(k - NBUF)
            gather(k)
        j = k - (NBUF - 1)
        if 0 <= j < NSTEP:
            gather_wait(j)
            writeback(j)
    for j in range(NSTEP - NBUF, NSTEP):
        writeback_wait(j)


def kernel(inputs, W0, W1, W2, W3):
    out4 = _co_embed(inputs.T.reshape(-1), W0, W1, W2, W3)
    return out4.transpose(0, 2, 1, 3).reshape(BATCH, NUM_FIELDS * ATTR_DIM)


# speedup vs baseline: 5.3114x; 1.0145x over previous
import functools

import jax
import jax.numpy as jnp
from jax import lax
from jax.experimental import pallas as pl
from jax.experimental.pallas import tpu as pltpu
from jax.experimental.pallas import tpu_sc as plsc

BATCH = 16384
NUM_FIELDS = 4
ATTR_DIM = 64

_info = plsc.get_sparse_core_info()
NC, NS, L = _info.num_cores, _info.num_subcores, _info.num_lanes
NW = NC * NS
BW = BATCH // NW
CHUNK = 128
NCHUNK = BW // CHUNK
NSTEP = NUM_FIELDS * NCHUNK
NBUF = 4

_mesh = plsc.VectorSubcoreMesh(core_axis_name="c", subcore_axis_name="s")


@functools.partial(
    pl.kernel,
    mesh=_mesh,
    compiler_params=pltpu.CompilerParams(use_tc_tiling_on_sc=False),
    out_type=jax.ShapeDtypeStruct((BATCH // 8, 2, 8, 128), jnp.float32),
    scratch_types=[
        pltpu.VMEM((NSTEP, CHUNK), jnp.int32),
        pltpu.VMEM((NBUF, CHUNK, ATTR_DIM), jnp.float32),
        pltpu.SemaphoreType.DMA,
        pltpu.SemaphoreType.DMA,
        pltpu.SemaphoreType.DMA,
        pltpu.SemaphoreType.DMA,
        pltpu.SemaphoreType.DMA,
        pltpu.SemaphoreType.DMA,
        pltpu.SemaphoreType.DMA,
        pltpu.SemaphoreType.DMA,
        pltpu.SemaphoreType.DMA,
    ],
)
def _co_embed(idx, w0, w1, w2, w3, out, idx_v, rows,
              isem, g0, g1, g2, g3, o0, o1, o2, o3):
    wid = lax.axis_index("s") * NC + lax.axis_index("c")
    base = wid * BW
    tables = [w0, w1, w2, w3]
    gsem = [g0, g1, g2, g3]
    osem = [o0, o1, o2, o3]

    def idx_src(k):
        f, c = k // NCHUNK, k % NCHUNK
        return idx.at[pl.ds(f * BATCH + base + c * CHUNK, CHUNK)]

    for k in range(NSTEP):
        pltpu.async_copy(idx_src(k), idx_v.at[k], isem)
    for k in range(NSTEP):
        pltpu.make_async_copy(idx_src(k), idx_v.at[k], isem).wait()

    def out_dst(k, t):
        f, c = k // NCHUNK, k % NCHUNK
        tile0 = (base + c * CHUNK) // 8
        return out.at[tile0 + t, f // 2, :,
                      pl.ds((f % 2) * ATTR_DIM, ATTR_DIM)]

    def gather(k):
        f = k // NCHUNK
        pltpu.async_copy(tables[f].at[idx_v.at[k]], rows.at[k % NBUF],
                         gsem[k % NBUF])

    def gather_wait(k):
        f = k // NCHUNK
        pltpu.make_async_copy(tables[f].at[idx_v.at[k]], rows.at[k % NBUF],
                              gsem[k % NBUF]).wait()

    def writeback(k):
        def body(t, _):
            pltpu.async_copy(rows.at[k % NBUF, pl.ds(t * 8, 8)],
                             out_dst(k, t), osem[k % NBUF])
            return _
        lax.fori_loop(0, CHUNK // 8, body, 0)

    def writeback_wait(k):
        def body(t, _):
            pltpu.make_async_copy(rows.at[k % NBUF, pl.ds(t * 8, 8)],
                                  out_dst(k, t), osem[k % NBUF]).wait()
            return _
        lax.fori_loop(0, CHUNK // 8, body, 0)

    for k in range(NSTEP + NBUF - 1):
        if k < NSTEP:
            if k >= NBUF:
                writeback_wait(k - NBUF)
            gather(k)
        j = k - (NBUF - 1)
        if 0 <= j < NSTEP:
            gather_wait(j)
            writeback(j)
    for j in range(NSTEP - NBUF, NSTEP):
        writeback_wait(j)


def kernel(inputs, W0, W1, W2, W3):
    out4 = _co_embed(inputs.T.reshape(-1), W0, W1, W2, W3)
    return out4.transpose(0, 2, 1, 3).reshape(BATCH, NUM_FIELDS * ATTR_DIM)


# speedup vs baseline: 5.6318x; 1.0603x over previous
import functools

import jax
import jax.numpy as jnp
from jax import lax
from jax.experimental import pallas as pl
from jax.experimental.pallas import tpu as pltpu
from jax.experimental.pallas import tpu_sc as plsc

BATCH = 16384
NUM_FIELDS = 4
ATTR_DIM = 64
VOCAB = 1000

_info = plsc.get_sparse_core_info()
NC, NS, L = _info.num_cores, _info.num_subcores, _info.num_lanes
NW = NC * NS
BW = BATCH // NW
CHUNK = 128
NCHUNK = BW // CHUNK
NSTEP = NUM_FIELDS * NCHUNK
NBUF = 4

_mesh = plsc.VectorSubcoreMesh(core_axis_name="c", subcore_axis_name="s")


@functools.partial(
    pl.kernel,
    mesh=_mesh,
    compiler_params=pltpu.CompilerParams(use_tc_tiling_on_sc=False),
    out_type=jax.ShapeDtypeStruct((BATCH // 8, 2, 8, 128), jnp.float32),
    scratch_types=[
        pltpu.VMEM((NSTEP, CHUNK), jnp.int32),
        pltpu.VMEM((NBUF, CHUNK, ATTR_DIM), jnp.float32),
        pltpu.SemaphoreType.DMA,
        pltpu.SemaphoreType.DMA,
        pltpu.SemaphoreType.DMA,
        pltpu.SemaphoreType.DMA,
        pltpu.SemaphoreType.DMA,
        pltpu.SemaphoreType.DMA,
        pltpu.SemaphoreType.DMA,
        pltpu.SemaphoreType.DMA,
        pltpu.SemaphoreType.DMA,
    ],
)
def _co_embed(idx, wall, out, idx_v, rows,
              isem, g0, g1, g2, g3, o0, o1, o2, o3):
    wid = lax.axis_index("s") * NC + lax.axis_index("c")
    base = wid * BW
    gsem = [g0, g1, g2, g3]
    osem = [o0, o1, o2, o3]

    def idx_src(k):
        f, c = k // NCHUNK, k % NCHUNK
        return idx.at[pl.ds(f * BATCH + base + c * CHUNK, CHUNK)]

    for k in range(NSTEP):
        pltpu.async_copy(idx_src(k), idx_v.at[k], isem)
    lane = lax.iota(jnp.int32, L)
    for k in range(NSTEP):
        pltpu.make_async_copy(idx_src(k), idx_v.at[k], isem).wait()
        f = k // NCHUNK
        if f:
            for g in range(CHUNK // L):
                sl = pl.ds(g * L, L)
                idx_v[k, sl] = idx_v[k, sl] + f * VOCAB

    def out_dst(k, t):
        f, c = k // NCHUNK, k % NCHUNK
        tile0 = (base + c * CHUNK) // 8
        return out.at[tile0 + t, f // 2, :,
                      pl.ds((f % 2) * ATTR_DIM, ATTR_DIM)]

    def gather(k):
        pltpu.async_copy(wall.at[idx_v.at[k]], rows.at[k % NBUF],
                         gsem[k % NBUF])

    def gather_wait(k):
        pltpu.make_async_copy(wall.at[idx_v.at[k]], rows.at[k % NBUF],
                              gsem[k % NBUF]).wait()

    def writeback(k):
        def body(t, _):
            pltpu.async_copy(rows.at[k % NBUF, pl.ds(t * 8, 8)],
                             out_dst(k, t), osem[k % NBUF])
            return _
        lax.fori_loop(0, CHUNK // 8, body, 0)

    def writeback_wait(k):
        def body(t, _):
            pltpu.make_async_copy(rows.at[k % NBUF, pl.ds(t * 8, 8)],
                                  out_dst(k, t), osem[k % NBUF]).wait()
            return _
        lax.fori_loop(0, CHUNK // 8, body, 0)

    for k in range(NSTEP + NBUF - 1):
        if k < NSTEP:
            if k >= NBUF:
                writeback_wait(k - NBUF)
            gather(k)
        j = k - (NBUF - 1)
        if 0 <= j < NSTEP:
            gather_wait(j)
            writeback(j)
    for j in range(NSTEP - NBUF, NSTEP):
        writeback_wait(j)


def kernel(inputs, W0, W1, W2, W3):
    wall = jnp.concatenate([W0, W1, W2, W3], axis=0)
    out4 = _co_embed(inputs.T.reshape(-1), wall)
    return out4.transpose(0, 2, 1, 3).reshape(BATCH, NUM_FIELDS * ATTR_DIM)


# speedup vs baseline: 5.7377x; 1.0188x over previous
import functools

import jax
import jax.numpy as jnp
from jax import lax
from jax.experimental import pallas as pl
from jax.experimental.pallas import tpu as pltpu
from jax.experimental.pallas import tpu_sc as plsc

BATCH = 16384
NUM_FIELDS = 4
ATTR_DIM = 64
VOCAB = 1000

_info = plsc.get_sparse_core_info()
NC, NS, L = _info.num_cores, _info.num_subcores, _info.num_lanes
NW = NC * NS
BW = BATCH // NW
CHUNK = 128
NCHUNK = BW // CHUNK
NSTEP = NUM_FIELDS * NCHUNK
NBUF = 8

_mesh = plsc.VectorSubcoreMesh(core_axis_name="c", subcore_axis_name="s")


@functools.partial(
    pl.kernel,
    mesh=_mesh,
    compiler_params=pltpu.CompilerParams(use_tc_tiling_on_sc=False),
    out_type=jax.ShapeDtypeStruct((BATCH // 8, 2, 8, 128), jnp.float32),
    scratch_types=[
        pltpu.VMEM((NSTEP, CHUNK), jnp.int32),
        pltpu.VMEM((NBUF, CHUNK, ATTR_DIM), jnp.float32),
        pltpu.SemaphoreType.DMA,
        pltpu.SemaphoreType.DMA,
        pltpu.SemaphoreType.DMA,
        pltpu.SemaphoreType.DMA,
        pltpu.SemaphoreType.DMA,
        pltpu.SemaphoreType.DMA,
        pltpu.SemaphoreType.DMA,
        pltpu.SemaphoreType.DMA,
        pltpu.SemaphoreType.DMA,
        pltpu.SemaphoreType.DMA,
        pltpu.SemaphoreType.DMA,
        pltpu.SemaphoreType.DMA,
        pltpu.SemaphoreType.DMA,
        pltpu.SemaphoreType.DMA,
        pltpu.SemaphoreType.DMA,
        pltpu.SemaphoreType.DMA,
        pltpu.SemaphoreType.DMA,
    ],
)
def _co_embed(idx, wall, out, idx_v, rows, isem,
              g0, g1, g2, g3, g4, g5, g6, g7,
              o0, o1, o2, o3, o4, o5, o6, o7):
    wid = lax.axis_index("s") * NC + lax.axis_index("c")
    base = wid * BW
    gsem = [g0, g1, g2, g3, g4, g5, g6, g7]
    osem = [o0, o1, o2, o3, o4, o5, o6, o7]

    def idx_src(k):
        f, c = k // NCHUNK, k % NCHUNK
        return idx.at[pl.ds(f * BATCH + base + c * CHUNK, CHUNK)]

    for k in range(NSTEP):
        pltpu.async_copy(idx_src(k), idx_v.at[k], isem)
    lane = lax.iota(jnp.int32, L)
    for k in range(NSTEP):
        pltpu.make_async_copy(idx_src(k), idx_v.at[k], isem).wait()
        f = k // NCHUNK
        if f:
            for g in range(CHUNK // L):
                sl = pl.ds(g * L, L)
                idx_v[k, sl] = idx_v[k, sl] + f * VOCAB

    def out_dst(k, t):
        f, c = k // NCHUNK, k % NCHUNK
        tile0 = (base + c * CHUNK) // 8
        return out.at[tile0 + t, f // 2, :,
                      pl.ds((f % 2) * ATTR_DIM, ATTR_DIM)]

    def gather(k):
        pltpu.async_copy(wall.at[idx_v.at[k]], rows.at[k % NBUF],
                         gsem[k % NBUF])

    def gather_wait(k):
        pltpu.make_async_copy(wall.at[idx_v.at[k]], rows.at[k % NBUF],
                              gsem[k % NBUF]).wait()

    def writeback(k):
        def body(t, _):
            pltpu.async_copy(rows.at[k % NBUF, pl.ds(t * 8, 8)],
                             out_dst(k, t), osem[k % NBUF])
            return _
        lax.fori_loop(0, CHUNK // 8, body, 0)

    def writeback_wait(k):
        def body(t, _):
            pltpu.make_async_copy(rows.at[k % NBUF, pl.ds(t * 8, 8)],
                                  out_dst(k, t), osem[k % NBUF]).wait()
            return _
        lax.fori_loop(0, CHUNK // 8, body, 0)

    for k in range(NSTEP + NBUF - 1):
        if k < NSTEP:
            if k >= NBUF:
                writeback_wait(k - NBUF)
            gather(k)
        j = k - (NBUF - 1)
        if 0 <= j < NSTEP:
            gather_wait(j)
            writeback(j)
    for j in range(NSTEP - NBUF, NSTEP):
        writeback_wait(j)


def kernel(inputs, W0, W1, W2, W3):
    wall = jnp.concatenate([W0, W1, W2, W3], axis=0)
    out4 = _co_embed(inputs.T.reshape(-1), wall)
    return out4.transpose(0, 2, 1, 3).reshape(BATCH, NUM_FIELDS * ATTR_DIM)


# speedup vs baseline: 5.7788x; 1.0072x over previous
import functools

import jax
import jax.numpy as jnp
from jax import lax
from jax.experimental import pallas as pl
from jax.experimental.pallas import tpu as pltpu
from jax.experimental.pallas import tpu_sc as plsc

BATCH = 16384
NUM_FIELDS = 4
ATTR_DIM = 64
VOCAB = 1000

_info = plsc.get_sparse_core_info()
NC, NS, L = _info.num_cores, _info.num_subcores, _info.num_lanes
NW = NC * NS
BW = BATCH // NW
CHUNK = 256
NCHUNK = BW // CHUNK
NSTEP = NUM_FIELDS * NCHUNK
NBUF = 4

_mesh = plsc.VectorSubcoreMesh(core_axis_name="c", subcore_axis_name="s")


@functools.partial(
    pl.kernel,
    mesh=_mesh,
    compiler_params=pltpu.CompilerParams(use_tc_tiling_on_sc=False),
    out_type=jax.ShapeDtypeStruct((BATCH // 8, 2, 8, 128), jnp.float32),
    scratch_types=[
        pltpu.VMEM((NSTEP, CHUNK), jnp.int32),
        pltpu.VMEM((NBUF, CHUNK, ATTR_DIM), jnp.float32),
        pltpu.SemaphoreType.DMA,
        pltpu.SemaphoreType.DMA,
        pltpu.SemaphoreType.DMA,
        pltpu.SemaphoreType.DMA,
        pltpu.SemaphoreType.DMA,
        pltpu.SemaphoreType.DMA,
        pltpu.SemaphoreType.DMA,
        pltpu.SemaphoreType.DMA,
        pltpu.SemaphoreType.DMA,
        pltpu.SemaphoreType.DMA,
        pltpu.SemaphoreType.DMA,
        pltpu.SemaphoreType.DMA,
        pltpu.SemaphoreType.DMA,
        pltpu.SemaphoreType.DMA,
        pltpu.SemaphoreType.DMA,
        pltpu.SemaphoreType.DMA,
        pltpu.SemaphoreType.DMA,
    ],
)
def _co_embed(idx, wall, out, idx_v, rows, isem,
              g0, g1, g2, g3, g4, g5, g6, g7,
              o0, o1, o2, o3, o4, o5, o6, o7):
    wid = lax.axis_index("s") * NC + lax.axis_index("c")
    base = wid * BW
    gsem = [g0, g1, g2, g3, g4, g5, g6, g7]
    osem = [o0, o1, o2, o3, o4, o5, o6, o7]

    def idx_src(k):
        f, c = k // NCHUNK, k % NCHUNK
        return idx.at[pl.ds(f * BATCH + base + c * CHUNK, CHUNK)]

    for k in range(NSTEP):
        pltpu.async_copy(idx_src(k), idx_v.at[k], isem)
    lane = lax.iota(jnp.int32, L)
    for k in range(NSTEP):
        pltpu.make_async_copy(idx_src(k), idx_v.at[k], isem).wait()
        f = k // NCHUNK
        if f:
            for g in range(CHUNK // L):
                sl = pl.ds(g * L, L)
                idx_v[k, sl] = idx_v[k, sl] + f * VOCAB

    def out_dst(k, t):
        f, c = k // NCHUNK, k % NCHUNK
        tile0 = (base + c * CHUNK) // 8
        return out.at[tile0 + t, f // 2, :,
                      pl.ds((f % 2) * ATTR_DIM, ATTR_DIM)]

    def gather(k):
        pltpu.async_copy(wall.at[idx_v.at[k]], rows.at[k % NBUF],
                         gsem[k % NBUF])

    def gather_wait(k):
        pltpu.make_async_copy(wall.at[idx_v.at[k]], rows.at[k % NBUF],
                              gsem[k % NBUF]).wait()

    def writeback(k):
        def body(t, _):
            pltpu.async_copy(rows.at[k % NBUF, pl.ds(t * 8, 8)],
                             out_dst(k, t), osem[k % NBUF])
            return _
        lax.fori_loop(0, CHUNK // 8, body, 0)

    def writeback_wait(k):
        def body(t, _):
            pltpu.make_async_copy(rows.at[k % NBUF, pl.ds(t * 8, 8)],
                                  out_dst(k, t), osem[k % NBUF]).wait()
            return _
        lax.fori_loop(0, CHUNK // 8, body, 0)

    for k in range(NSTEP + NBUF - 1):
        if k < NSTEP:
            if k >= NBUF:
                writeback_wait(k - NBUF)
            gather(k)
        j = k - (NBUF - 1)
        if 0 <= j < NSTEP:
            gather_wait(j)
            writeback(j)
    for j in range(NSTEP - NBUF, NSTEP):
        writeback_wait(j)


def kernel(inputs, W0, W1, W2, W3):
    wall = jnp.concatenate([W0, W1, W2, W3], axis=0)
    out4 = _co_embed(inputs.T.reshape(-1), wall)
    return out4.transpose(0, 2, 1, 3).reshape(BATCH, NUM_FIELDS * ATTR_DIM)


# speedup vs baseline: 5.8665x; 1.0152x over previous
import functools

import jax
import jax.numpy as jnp
from jax import lax
from jax.experimental import pallas as pl
from jax.experimental.pallas import tpu as pltpu
from jax.experimental.pallas import tpu_sc as plsc

BATCH = 16384
NUM_FIELDS = 4
ATTR_DIM = 64
VOCAB = 1000

_info = plsc.get_sparse_core_info()
NC, NS, L = _info.num_cores, _info.num_subcores, _info.num_lanes
NW = NC * NS
BW = BATCH // NW
CHUNK = 256
NCHUNK = BW // CHUNK
NSTEP = NUM_FIELDS * NCHUNK
NBUF = 4

_mesh = plsc.VectorSubcoreMesh(core_axis_name="c", subcore_axis_name="s")


@functools.partial(
    pl.kernel,
    mesh=_mesh,
    compiler_params=pltpu.CompilerParams(use_tc_tiling_on_sc=False),
    out_type=jax.ShapeDtypeStruct((BATCH // 8, 2, 8, 128), jnp.float32),
    scratch_types=[
        pltpu.VMEM((NSTEP, CHUNK), jnp.int32),
        pltpu.VMEM((NBUF, CHUNK, ATTR_DIM), jnp.float32),
        pltpu.SemaphoreType.DMA,
        pltpu.SemaphoreType.DMA((NBUF,)),
        pltpu.SemaphoreType.DMA((NBUF,)),
    ],
)
def _co_embed(idx, wall, out, idx_v, rows, isem, gsem, osem):
    wid = lax.axis_index("s") * NC + lax.axis_index("c")
    base = wid * BW

    def idx_src(k):
        f = k // NCHUNK
        c = lax.rem(k, NCHUNK) if not isinstance(k, int) else k % NCHUNK
        return idx.at[pl.ds(f * BATCH + base + c * CHUNK, CHUNK)]

    def stage(k, _):
        pltpu.async_copy(idx_src(k), idx_v.at[k], isem)
        return _

    def stage_wait(k, _):
        pltpu.make_async_copy(idx_src(k), idx_v.at[k], isem).wait()
        return _

    lax.fori_loop(0, NSTEP, stage, 0)
    lax.fori_loop(0, NSTEP, stage_wait, 0)

    def slot(k):
        return k % NBUF if isinstance(k, int) else lax.rem(k, NBUF)

    def gather(k):
        pltpu.async_copy(wall.at[idx_v.at[k]], rows.at[slot(k)],
                         gsem.at[slot(k)])

    def gather_wait(k):
        pltpu.make_async_copy(wall.at[idx_v.at[k]], rows.at[slot(k)],
                              gsem.at[slot(k)]).wait()

    def wb_dst(k, t):
        f = k // NCHUNK
        c = lax.rem(k, NCHUNK) if not isinstance(k, int) else k % NCHUNK
        tile0 = (base + c * CHUNK) // 8
        return out.at[tile0 + t, f // 2, :,
                      pl.ds((f % 2) * ATTR_DIM, ATTR_DIM)]

    def writeback(k):
        def body(t, _):
            pltpu.async_copy(rows.at[slot(k), pl.ds(t * 8, 8)],
                             wb_dst(k, t), osem.at[slot(k)])
            return _
        lax.fori_loop(0, CHUNK // 8, body, 0)

    def writeback_wait(k):
        def body(t, _):
            pltpu.make_async_copy(rows.at[slot(k), pl.ds(t * 8, 8)],
                                  wb_dst(k, t), osem.at[slot(k)]).wait()
            return _
        lax.fori_loop(0, CHUNK // 8, body, 0)

    def pipe(k, _):
        @pl.when(k >= NBUF)
        def _w():
            writeback_wait(k - NBUF)
        gather(k)

        @pl.when(k >= NBUF - 1)
        def _g():
            gather_wait(k - (NBUF - 1))
            writeback(k - (NBUF - 1))
        return _

    lax.fori_loop(0, NSTEP, pipe, 0)
    for j in range(NSTEP - NBUF + 1, NSTEP):
        gather_wait(j)
        writeback(j)
    for j in range(NSTEP - NBUF, NSTEP):
        writeback_wait(j)


def kernel(inputs, W0, W1, W2, W3):
    wall = jnp.concatenate([W0, W1, W2, W3], axis=0)
    shifted = inputs + jnp.arange(NUM_FIELDS, dtype=inputs.dtype) * VOCAB
    out4 = _co_embed(shifted.T.reshape(-1), wall)
    return out4.transpose(0, 2, 1, 3).reshape(BATCH, NUM_FIELDS * ATTR_DIM)


# speedup vs baseline: 5.9474x; 1.0138x over previous
import functools

import jax
import jax.numpy as jnp
from jax import lax
from jax.experimental import pallas as pl
from jax.experimental.pallas import tpu as pltpu
from jax.experimental.pallas import tpu_sc as plsc

BATCH = 16384
NUM_FIELDS = 4
ATTR_DIM = 64
VOCAB = 1000

_info = plsc.get_sparse_core_info()
NC, NS, L = _info.num_cores, _info.num_subcores, _info.num_lanes
NW = NC * NS
BW = BATCH // NW
CHUNK = 256
NCHUNK = BW // CHUNK
NSTEP = NUM_FIELDS * NCHUNK
NBUF = 6

_mesh = plsc.VectorSubcoreMesh(core_axis_name="c", subcore_axis_name="s")


@functools.partial(
    pl.kernel,
    mesh=_mesh,
    compiler_params=pltpu.CompilerParams(use_tc_tiling_on_sc=False),
    out_type=jax.ShapeDtypeStruct((BATCH // 8, 2, 8, 128), jnp.float32),
    scratch_types=[
        pltpu.VMEM((NSTEP, CHUNK), jnp.int32),
        pltpu.VMEM((NBUF, CHUNK, ATTR_DIM), jnp.float32),
        pltpu.SemaphoreType.DMA,
        pltpu.SemaphoreType.DMA((NBUF,)),
        pltpu.SemaphoreType.DMA((NBUF,)),
    ],
)
def _co_embed(idx, wall, out, idx_v, rows, isem, gsem, osem):
    wid = lax.axis_index("s") * NC + lax.axis_index("c")
    base = wid * BW

    def idx_src(k):
        f = k // NCHUNK
        c = lax.rem(k, NCHUNK) if not isinstance(k, int) else k % NCHUNK
        return idx.at[pl.ds(f * BATCH + base + c * CHUNK, CHUNK)]

    def stage(k, _):
        pltpu.async_copy(idx_src(k), idx_v.at[k], isem)
        return _

    def stage_wait(k, _):
        pltpu.make_async_copy(idx_src(k), idx_v.at[k], isem).wait()
        return _

    lax.fori_loop(0, NSTEP, stage, 0)

    def slot(k):
        return k % NBUF if isinstance(k, int) else lax.rem(k, NBUF)

    def gather(k):
        stage_wait(k, 0)
        pltpu.async_copy(wall.at[idx_v.at[k]], rows.at[slot(k)],
                         gsem.at[slot(k)])

    def gather_wait(k):
        pltpu.make_async_copy(wall.at[idx_v.at[k]], rows.at[slot(k)],
                              gsem.at[slot(k)]).wait()

    def wb_dst(k, t):
        f = k // NCHUNK
        c = lax.rem(k, NCHUNK) if not isinstance(k, int) else k % NCHUNK
        tile0 = (base + c * CHUNK) // 8
        return out.at[tile0 + t, f // 2, :,
                      pl.ds((f % 2) * ATTR_DIM, ATTR_DIM)]

    def writeback(k):
        def body(t, _):
            pltpu.async_copy(rows.at[slot(k), pl.ds(t * 8, 8)],
                             wb_dst(k, t), osem.at[slot(k)])
            return _
        lax.fori_loop(0, CHUNK // 8, body, 0)

    def writeback_wait(k):
        def body(t, _):
            pltpu.make_async_copy(rows.at[slot(k), pl.ds(t * 8, 8)],
                                  wb_dst(k, t), osem.at[slot(k)]).wait()
            return _
        lax.fori_loop(0, CHUNK // 8, body, 0)

    def pipe(k, _):
        @pl.when(k >= NBUF)
        def _w():
            writeback_wait(k - NBUF)
        gather(k)

        @pl.when(k >= NBUF - 1)
        def _g():
            gather_wait(k - (NBUF - 1))
            writeback(k - (NBUF - 1))
        return _

    lax.fori_loop(0, NSTEP, pipe, 0)
    for j in range(NSTEP - NBUF + 1, NSTEP):
        gather_wait(j)
        writeback(j)
    for j in range(NSTEP - NBUF, NSTEP):
        writeback_wait(j)


def kernel(inputs, W0, W1, W2, W3):
    wall = jnp.concatenate([W0, W1, W2, W3], axis=0)
    shifted = inputs + jnp.arange(NUM_FIELDS, dtype=inputs.dtype) * VOCAB
    out4 = _co_embed(shifted.T.reshape(-1), wall)
    return out4.transpose(0, 2, 1, 3).reshape(BATCH, NUM_FIELDS * ATTR_DIM)
